# baseline (speedup 1.0000x reference)
import functools

import jax
import jax.numpy as jnp
from jax import lax
from jax.experimental import pallas as pl
from jax.experimental.pallas import tpu as pltpu
from jax.experimental.pallas import tpu_sc as plsc

S = 5000
W = 100
F = 8
D = 32

NC = 2
NS = 16
NW = NC * NS
CH = 152
REM = 8
NREM = (S - NW * CH) // REM
CW = 16
SF = F + W


def _colsum_sc_body(state_hbm, out_hbm, buf, rembuf, accv):
    wid = lax.axis_index("s") * NC + lax.axis_index("c")
    base = wid * CH
    pltpu.sync_copy(state_hbm.at[pl.ds(base, CH)], buf)

    def body(i, a):
        return a + buf[i, 0:CW]

    acc = lax.fori_loop(0, CH, body, jnp.zeros((CW,), jnp.float32))

    rbase = NW * CH + REM * jnp.minimum(wid, NREM - 1)
    pltpu.sync_copy(state_hbm.at[pl.ds(rbase, REM)], rembuf)
    racc = jnp.zeros((CW,), jnp.float32)
    for i in range(REM):
        racc = racc + rembuf[i, 0:CW]
    keep = jnp.where(wid < NREM, 1.0, 0.0).astype(jnp.float32)
    accv[...] = acc + racc * keep
    pltpu.sync_copy(accv, out_hbm.at[wid])


def _mlp_body(part_ref, we_ref, be_ref, w1_ref, b1_ref, w2_ref, b2_ref,
              wd_ref, bd_ref, out_ref):
    total = jnp.sum(part_ref[...], axis=0, keepdims=True)
    mean_r = total[:, 0:F] * (1.0 / S)

    mw = jnp.dot(mean_r, we_ref[...],
                 preferred_element_type=jnp.float32) + be_ref[...]

    h1w = jax.nn.relu(jnp.dot(mw, w1_ref[...],
                              preferred_element_type=jnp.float32) + b1_ref[...])
    h1s = jax.nn.relu(b1_ref[...])
    h2s = jnp.dot(h1w, w2_ref[...],
                  preferred_element_type=jnp.float32) + b2_ref[...]
    h2w = jnp.dot(h1s, w2_ref[...],
                  preferred_element_type=jnp.float32) + b2_ref[...]

    dec_in = jnp.concatenate([h2s, h2w], axis=1)
    score = jnp.dot(dec_in, wd_ref[...],
                    preferred_element_type=jnp.float32) + bd_ref[...]
    srow = jnp.broadcast_to(score, (1, W))
    m = jnp.max(srow, axis=1, keepdims=True)
    e = jnp.exp(srow - m)
    out_ref[...] = e / jnp.sum(e, axis=1, keepdims=True)


def kernel(state, W_embed, b_embed, W1, b1, W2, b2, W_dec, b_dec):
    mesh = plsc.VectorSubcoreMesh(core_axis_name="c", subcore_axis_name="s",
                                  num_cores=NC, num_subcores=NS)
    colsum = functools.partial(
        pl.kernel,
        out_type=jax.ShapeDtypeStruct((NW, CW), jnp.float32),
        mesh=mesh,
        scratch_types=[
            pltpu.VMEM((CH, SF), jnp.float32),
            pltpu.VMEM((REM, SF), jnp.float32),
            pltpu.VMEM((CW,), jnp.float32),
        ],
    )(_colsum_sc_body)
    partials = colsum(state)

    out = pl.pallas_call(
        _mlp_body,
        out_shape=jax.ShapeDtypeStruct((1, W), jnp.float32),
    )(partials,
      W_embed, b_embed.reshape(1, D),
      W1, b1.reshape(1, D),
      W2, b2.reshape(1, D),
      W_dec, b_dec.reshape(1, 1))
    return out.reshape(W)


# speedup vs baseline: 1.0506x; 1.0506x over previous
import functools

import jax
import jax.numpy as jnp
from jax import lax
from jax.experimental import pallas as pl
from jax.experimental.pallas import tpu as pltpu
from jax.experimental.pallas import tpu_sc as plsc

S = 5000
W = 100
F = 8
D = 32

NC = 2
NS = 16
NW = NC * NS
FL = S * F
CH = 1248
REM = FL - NW * CH
NV = CH // 16


def _colsum_sc_body(feats_hbm, out_hbm, buf, rembuf, accv):
    wid = lax.axis_index("s") * NC + lax.axis_index("c")
    base = wid * CH
    pltpu.sync_copy(feats_hbm.at[pl.ds(base, CH)], buf)

    def body(i, a):
        return a + buf[pl.ds(i * 16, 16)]

    acc = lax.fori_loop(0, NV, body, jnp.zeros((16,), jnp.float32))

    pltpu.sync_copy(feats_hbm.at[pl.ds(NW * CH, REM)], rembuf)
    racc = jnp.zeros((16,), jnp.float32)
    for i in range(REM // 16):
        racc = racc + rembuf[pl.ds(i * 16, 16)]
    keep = jnp.where(wid == 0, 1.0, 0.0).astype(jnp.float32)
    accv[...] = acc + racc * keep
    pltpu.sync_copy(accv, out_hbm.at[wid])


def _mlp_body(part_ref, we_ref, be_ref, w1_ref, b1_ref, w2_ref, b2_ref,
              wd_ref, bd_ref, out_ref):
    total = jnp.sum(part_ref[...], axis=0, keepdims=True)
    mean_r = (total[:, 0:F] + total[:, F:2 * F]) * (1.0 / S)

    mw = jnp.dot(mean_r, we_ref[...],
                 preferred_element_type=jnp.float32) + be_ref[...]

    h1w = jax.nn.relu(jnp.dot(mw, w1_ref[...],
                              preferred_element_type=jnp.float32) + b1_ref[...])
    h1s = jax.nn.relu(b1_ref[...])
    h2s = jnp.dot(h1w, w2_ref[...],
                  preferred_element_type=jnp.float32) + b2_ref[...]
    h2w = jnp.dot(h1s, w2_ref[...],
                  preferred_element_type=jnp.float32) + b2_ref[...]

    dec_in = jnp.concatenate([h2s, h2w], axis=1)
    score = jnp.dot(dec_in, wd_ref[...],
                    preferred_element_type=jnp.float32) + bd_ref[...]
    srow = jnp.broadcast_to(score, (1, W))
    m = jnp.max(srow, axis=1, keepdims=True)
    e = jnp.exp(srow - m)
    out_ref[...] = e / jnp.sum(e, axis=1, keepdims=True)


def kernel(state, W_embed, b_embed, W1, b1, W2, b2, W_dec, b_dec):
    feats = state[:, :F].reshape(FL)
    mesh = plsc.VectorSubcoreMesh(core_axis_name="c", subcore_axis_name="s",
                                  num_cores=NC, num_subcores=NS)
    colsum = functools.partial(
        pl.kernel,
        out_type=jax.ShapeDtypeStruct((NW, 16), jnp.float32),
        mesh=mesh,
        scratch_types=[
            pltpu.VMEM((CH,), jnp.float32),
            pltpu.VMEM((REM,), jnp.float32),
            pltpu.VMEM((16,), jnp.float32),
        ],
    )(_colsum_sc_body)
    partials = colsum(feats)

    out = pl.pallas_call(
        _mlp_body,
        out_shape=jax.ShapeDtypeStruct((1, W), jnp.float32),
    )(partials,
      W_embed, b_embed.reshape(1, D),
      W1, b1.reshape(1, D),
      W2, b2.reshape(1, D),
      W_dec, b_dec.reshape(1, 1))
    return out.reshape(W)
